# baseline (speedup 1.0000x reference)
import jax
import jax.numpy as jnp
from jax import lax
from jax.experimental import pallas as pl
from jax.experimental.pallas import tpu as pltpu
from jax.experimental.pallas import tpu_sc as plsc

NC = 2
NS = 16
L = 16
NW = NC * NS

B = 16384
D = 64
BPW = B // NW
GCHUNK = 128
NCHUNK = BPW // GCHUNK

_SC_PARAMS = pltpu.CompilerParams(use_tc_tiling_on_sc=False,
                                  skip_device_barrier=True)


def _wid():
    return lax.axis_index("s") * NC + lax.axis_index("c")


def _gather_body(idx_hbm, tab_hbm, out_hbm, idx_v, rows_v, sem):
    base = _wid() * BPW
    pltpu.sync_copy(idx_hbm.at[pl.ds(base, BPW)], idx_v)
    handles = []
    for j in range(NCHUNK):
        sl = pl.ds(j * GCHUNK, GCHUNK)
        handles.append(pltpu.async_copy(tab_hbm.at[idx_v.at[sl]], rows_v.at[sl], sem))
    for h in handles:
        h.wait()
    pltpu.sync_copy(rows_v, out_hbm.at[pl.ds(base, BPW)])


def _dot_body(u_hbm, i_hbm, out_hbm, u_rows, i_rows, out_v, semu, semi):
    base = _wid() * BPW
    hu = pltpu.async_copy(u_hbm.at[pl.ds(base, BPW)], u_rows, semu)
    hi = pltpu.async_copy(i_hbm.at[pl.ds(base, BPW)], i_rows, semi)
    hu.wait()
    hi.wait()

    lanes = lax.iota(jnp.int32, L)
    perms = {h: lanes ^ h for h in (8, 4, 2, 1)}
    masks = {h: (lanes & h) != 0 for h in (8, 4, 2, 1)}

    def lperm(v, h):
        return v.at[perms[h]].get(mode="promise_in_bounds", unique_indices=True)

    def group(g, _):
        vs = []
        for rl in range(L):
            r = g * L + rl
            acc = u_rows[r, pl.ds(0, L)] * i_rows[r, pl.ds(0, L)]
            for k in range(1, D // L):
                acc += u_rows[r, pl.ds(k * L, L)] * i_rows[r, pl.ds(k * L, L)]
            vs.append(acc)
        for h in (8, 4, 2, 1):
            half = len(vs) // 2
            vs = [jnp.where(masks[h],
                            vs[q + half] + lperm(vs[q + half], h),
                            vs[q] + lperm(vs[q], h))
                  for q in range(half)]
        out_v[pl.ds(g * L, L)] = vs[0]
        return 0

    lax.fori_loop(0, BPW // L, group, 0)
    pltpu.sync_copy(out_v, out_hbm.at[pl.ds(base, BPW)])


def _mesh():
    return plsc.VectorSubcoreMesh(core_axis_name="c", subcore_axis_name="s")


def _gather(idx, table):
    return pl.kernel(
        _gather_body,
        out_type=jax.ShapeDtypeStruct((B, D), jnp.float32),
        mesh=_mesh(),
        compiler_params=_SC_PARAMS,
        scratch_types=[
            pltpu.VMEM((BPW,), jnp.int32),
            pltpu.VMEM((BPW, D), jnp.float32),
            pltpu.SemaphoreType.DMA,
        ],
    )(idx, table)


@jax.jit
def _mf_dot(user_id, item_id, user_table, item_table):
    u_rows = _gather(user_id, user_table)
    i_rows = _gather(item_id, item_table)
    return pl.kernel(
        _dot_body,
        out_type=jax.ShapeDtypeStruct((B,), jnp.float32),
        mesh=_mesh(),
        compiler_params=_SC_PARAMS,
        scratch_types=[
            pltpu.VMEM((BPW, D), jnp.float32),
            pltpu.VMEM((BPW, D), jnp.float32),
            pltpu.VMEM((BPW,), jnp.float32),
            pltpu.SemaphoreType.DMA,
            pltpu.SemaphoreType.DMA,
        ],
    )(u_rows, i_rows)


def kernel(user_id, item_id, user_table, item_table):
    return _mf_dot(user_id, item_id, user_table, item_table)


# speedup vs baseline: 1.4898x; 1.4898x over previous
import jax
import jax.numpy as jnp
from jax import lax
from jax.experimental import pallas as pl
from jax.experimental.pallas import tpu as pltpu
from jax.experimental.pallas import tpu_sc as plsc

NC = 2
NS = 16
L = 16
NW = NC * NS

B = 16384
D = 64
NSEM = 4

B_TC = 4096
B_SC = B - B_TC
TC_CH = 512


def _sc_body(uid_hbm, iid_hbm, ut_hbm, it_hbm, out_hbm,
             uidx_v, iidx_v, u_rows, i_rows, out_v, *sems):
    usems = sems[:NSEM]
    isems = sems[NSEM:]
    bpw = B_SC // NW
    prows = bpw // 2
    base = lax.axis_index("s") * NC * bpw + lax.axis_index("c") * bpw
    wid = lax.axis_index("s") * NC + lax.axis_index("c")
    base = wid * bpw

    pltpu.sync_copy(uid_hbm.at[pl.ds(base, bpw)], uidx_v)
    pltpu.sync_copy(iid_hbm.at[pl.ds(base, bpw)], iidx_v)

    lanes = lax.iota(jnp.int32, L)
    perms = {h: lanes ^ h for h in (8, 4, 2, 1)}
    masks = {h: (lanes & h) != 0 for h in (8, 4, 2, 1)}

    def lperm(v, h):
        return v.at[perms[h]].get(mode="promise_in_bounds", unique_indices=True)

    for pp in range(2):
        pbase = pp * prows
        ng = prows // L
        gq = ng // NSEM

        for g in range(ng):
            q = g // gq
            uvec = uidx_v[pl.ds(pbase + g * L, L)]
            ivec = iidx_v[pl.ds(pbase + g * L, L)]
            for rl in range(L):
                r = g * L + rl
                pltpu.async_copy(ut_hbm.at[pl.ds(uvec[rl], 1)],
                                 u_rows.at[pl.ds(r, 1)], usems[q])
                pltpu.async_copy(it_hbm.at[pl.ds(ivec[rl], 1)],
                                 i_rows.at[pl.ds(r, 1)], isems[q])

        chunk = prows // NSEM
        for j in range(NSEM):
            sl = pl.ds(j * chunk, chunk)
            pltpu.make_async_copy(ut_hbm.at[pl.ds(0, chunk)], u_rows.at[sl], usems[j]).wait()
            pltpu.make_async_copy(it_hbm.at[pl.ds(0, chunk)], i_rows.at[sl], isems[j]).wait()

        def group(g, _):
            vs = []
            for rl in range(L):
                r = g * L + rl
                acc = u_rows[r, pl.ds(0, L)] * i_rows[r, pl.ds(0, L)]
                for k in range(1, D // L):
                    acc += u_rows[r, pl.ds(k * L, L)] * i_rows[r, pl.ds(k * L, L)]
                vs.append(acc)
            for h in (8, 4, 2, 1):
                half = len(vs) // 2
                vs = [jnp.where(masks[h],
                                vs[q2 + half] + lperm(vs[q2 + half], h),
                                vs[q2] + lperm(vs[q2], h))
                      for q2 in range(half)]
            out_v[pl.ds(pbase + g * L, L)] = vs[0]
            return 0

        lax.fori_loop(0, ng, group, 0)

    pltpu.sync_copy(out_v, out_hbm.at[pl.ds(base, bpw)])


def _sc_part(uid, iid, ut, it):
    bpw = B_SC // NW
    mesh = plsc.VectorSubcoreMesh(core_axis_name="c", subcore_axis_name="s")
    return pl.kernel(
        _sc_body,
        out_type=jax.ShapeDtypeStruct((B_SC,), jnp.float32),
        mesh=mesh,
        scratch_types=[
            pltpu.VMEM((bpw,), jnp.int32),
            pltpu.VMEM((bpw,), jnp.int32),
            pltpu.VMEM((bpw // 2, D), jnp.float32),
            pltpu.VMEM((bpw // 2, D), jnp.float32),
            pltpu.VMEM((bpw,), jnp.float32),
        ] + [pltpu.SemaphoreType.DMA] * (2 * NSEM),
    )(uid, iid, ut, it)


def _tc_body(uid_s, iid_s, ut_hbm, it_hbm, out_v, u_v, i_v, semu, semi):
    def fire(r, _):
        pltpu.make_async_copy(ut_hbm.at[pl.ds(uid_s[r], 1)],
                              u_v.at[pl.ds(r, 1)], semu).start()
        pltpu.make_async_copy(it_hbm.at[pl.ds(iid_s[r], 1)],
                              i_v.at[pl.ds(r, 1)], semi).start()
        return 0

    lax.fori_loop(0, TC_CH, fire, 0)

    def drain(r, _):
        pltpu.make_async_copy(ut_hbm.at[pl.ds(uid_s[r], 1)],
                              u_v.at[pl.ds(r, 1)], semu).wait()
        pltpu.make_async_copy(it_hbm.at[pl.ds(iid_s[r], 1)],
                              i_v.at[pl.ds(r, 1)], semi).wait()
        return 0

    lax.fori_loop(0, TC_CH, drain, 0)

    out_v[...] = jnp.sum(u_v[...] * i_v[...], axis=1)


def _tc_part(uid, iid, ut, it):
    nch = B_TC // TC_CH
    return pl.pallas_call(
        _tc_body,
        grid=(nch,),
        in_specs=[
            pl.BlockSpec((TC_CH,), lambda i: (i,), memory_space=pltpu.SMEM),
            pl.BlockSpec((TC_CH,), lambda i: (i,), memory_space=pltpu.SMEM),
            pl.BlockSpec(memory_space=pltpu.HBM),
            pl.BlockSpec(memory_space=pltpu.HBM),
        ],
        out_specs=pl.BlockSpec((TC_CH,), lambda i: (i,)),
        out_shape=jax.ShapeDtypeStruct((B_TC,), jnp.float32),
        scratch_shapes=[
            pltpu.VMEM((TC_CH, D), jnp.float32),
            pltpu.VMEM((TC_CH, D), jnp.float32),
            pltpu.SemaphoreType.DMA,
            pltpu.SemaphoreType.DMA,
        ],
    )(uid, iid, ut, it)


@jax.jit
def _mf_dot(user_id, item_id, user_table, item_table):
    out_sc = _sc_part(user_id[:B_SC], item_id[:B_SC], user_table, item_table)
    out_tc = _tc_part(user_id[B_SC:], item_id[B_SC:], user_table, item_table)
    return jnp.concatenate([out_sc, out_tc])


def kernel(user_id, item_id, user_table, item_table):
    return _mf_dot(user_id, item_id, user_table, item_table)
